# baseline (speedup 1.0000x reference)
import functools

import jax
import jax.numpy as jnp
from jax import lax
from jax.experimental import pallas as pl
from jax.experimental.pallas import tpu as pltpu
from jax.experimental.pallas import tpu_sc as plsc

NB = 2
L = 16


@functools.lru_cache(maxsize=None)
def _make_sc_lookup(batch, seqlen, embed, vocab):
    info = plsc.get_sparse_core_info()
    nw = info.num_cores * info.num_subcores
    assert batch % nw == 0 and vocab % 2 == 0 and embed % L == 0
    chunks = batch // nw
    assert chunks % NB == 0 and chunks >= NB
    ngrp = (seqlen + L - 1) // L
    npad = ngrp * L
    mesh = plsc.VectorSubcoreMesh(core_axis_name="c", subcore_axis_name="s")

    @functools.partial(
        pl.kernel,
        mesh=mesh,
        compiler_params=pltpu.CompilerParams(needs_layout_passes=False),
        out_type=jax.ShapeDtypeStruct((batch, seqlen, embed), jnp.float32),
        scratch_types=(
            [pltpu.VMEM((npad, 2 * embed), jnp.float32) for _ in range(NB)]
            + [pltpu.VMEM((seqlen, embed), jnp.float32) for _ in range(NB)]
            + [pltpu.VMEM((npad,), jnp.int32) for _ in range(NB)]
            + [pltpu.VMEM((npad,), jnp.int32) for _ in range(NB)]
            + [pltpu.VMEM((seqlen, embed), jnp.float32)]
            + [pltpu.SemaphoreType.DMA for _ in range(3 * NB)]
        ),
    )
    def k(x_hbm, tok_hbm, pos_hbm, out_hbm, *scratch):
        pairs = scratch[:NB]
        rows = scratch[NB:2 * NB]
        idxs = scratch[2 * NB:3 * NB]
        pidxs = scratch[3 * NB:4 * NB]
        pos_v = scratch[4 * NB]
        isem = scratch[4 * NB + 1:4 * NB + 1 + NB]
        gsem = scratch[4 * NB + 1 + NB:4 * NB + 1 + 2 * NB]
        ssem = scratch[4 * NB + 1 + 2 * NB:]

        wid = lax.axis_index("s") * info.num_cores + lax.axis_index("c")
        row0 = wid * chunks

        pltpu.sync_copy(pos_hbm, pos_v)
        if npad > seqlen:
            for b in range(NB):
                pidxs[b][pl.ds(npad - L, L)] = jnp.zeros((L,), jnp.int32)

        def x_slice(i):
            return x_hbm.at[pl.ds((row0 + i) * seqlen, seqlen)]

        def fetch(i, b):
            @pl.when(i < chunks)
            def _():
                pltpu.async_copy(x_slice(i), idxs[b].at[pl.ds(0, seqlen)],
                                 isem[b])

        def gather(i, b):
            @pl.when(jnp.logical_and(i >= 0, i < chunks))
            def _():
                pltpu.make_async_copy(x_slice(i),
                                      idxs[b].at[pl.ds(0, seqlen)],
                                      isem[b]).wait()

                def to_pair(g, _):
                    rowv = g * L + lax.iota(jnp.int32, L)
                    live = rowv < seqlen
                    rowc = jnp.where(live, rowv, 0)
                    v = lax.shift_right_logical(
                        plsc.load_gather(idxs[b], [rowc]), 1)
                    plsc.store_scatter(pidxs[b], [rowc], v, mask=live)
                    return 0

                lax.fori_loop(0, ngrp, to_pair, 0)
                pltpu.async_copy(tok_hbm.at[pidxs[b]], pairs[b], gsem[b])

        def emit(i, b):
            @pl.when(jnp.logical_and(i >= 0, i < chunks))
            def _():
                pltpu.make_async_copy(tok_hbm.at[pidxs[b]], pairs[b],
                                      gsem[b]).wait()

                @pl.when(i >= NB)
                def _():
                    pltpu.make_async_copy(
                        rows[b], out_hbm.at[row0 + i - NB], ssem[b]).wait()

                def sel_grp(g, _):
                    rowv = g * L + lax.iota(jnp.int32, L)
                    live = rowv < seqlen
                    rowc = jnp.where(live, rowv, 0)
                    tokv = plsc.load_gather(idxs[b], [rowc])
                    offv = lax.shift_left(jnp.bitwise_and(tokv, 1), 6)
                    for c in range(embed):
                        cv = jnp.full((L,), c, jnp.int32)
                        val = plsc.load_gather(pairs[b], [rowc, offv + c])
                        pv = plsc.load_gather(pos_v, [rowc, cv])
                        plsc.store_scatter(rows[b], [rowc, cv],
                                           val + pv, mask=live)
                    return 0

                lax.fori_loop(0, ngrp, sel_grp, 0)
                pltpu.async_copy(rows[b], out_hbm.at[row0 + i], ssem[b])

        def visit_group(kk, _):
            for j in range(2 * NB):
                v = 2 * NB * kk + j - 2
                emit(v, j % NB)
                fetch(v + 2, j % NB)
                gather(v + 1, (j + 1) % NB)
            return 0

        lax.fori_loop(0, chunks // (2 * NB) + 1, visit_group, 0)

        for b in range(NB):
            pltpu.make_async_copy(rows[b], out_hbm.at[row0], ssem[b]).wait()

    return k


def kernel(x, token_table, pos_table):
    batch, seqlen = x.shape
    vocab, embed = token_table.shape
    k = _make_sc_lookup(batch, seqlen, embed, vocab)
    return k(x.reshape(-1).astype(jnp.int32),
             token_table.reshape(vocab // 2, 2 * embed), pos_table)


# speedup vs baseline: 3.3440x; 3.3440x over previous
import functools

import jax
import jax.numpy as jnp
from jax import lax
from jax.experimental import pallas as pl
from jax.experimental.pallas import tpu as pltpu
from jax.experimental.pallas import tpu_sc as plsc

NB = 4


@functools.lru_cache(maxsize=None)
def _make_sc_lookup(batch, seqlen, embed, vocab):
    info = plsc.get_sparse_core_info()
    nw = info.num_cores * info.num_subcores
    assert batch % nw == 0
    chunks = batch // nw
    assert chunks % NB == 0 and chunks >= NB
    mesh = plsc.VectorSubcoreMesh(core_axis_name="c", subcore_axis_name="s")

    @functools.partial(
        pl.kernel,
        mesh=mesh,
        compiler_params=pltpu.CompilerParams(use_tc_tiling_on_sc=False),
        out_type=jax.ShapeDtypeStruct((batch, seqlen, embed), jnp.float32),
        scratch_types=(
            [pltpu.VMEM((seqlen, embed), jnp.float32) for _ in range(NB)]
            + [pltpu.VMEM((seqlen,), jnp.int32) for _ in range(NB)]
            + [pltpu.VMEM_SHARED((seqlen, embed), jnp.float32)]
            + [pltpu.SemaphoreType.DMA for _ in range(4 * NB)]
        ),
    )
    def k(x_hbm, tok_hbm, pos_hbm, out_hbm, *scratch):
        rows = scratch[:NB]
        idxs = scratch[NB:2 * NB]
        pos_sh = scratch[2 * NB]
        isem = scratch[2 * NB + 1:2 * NB + 1 + NB]
        nsem = scratch[2 * NB + 1 + NB:2 * NB + 1 + 2 * NB]
        gsem = scratch[2 * NB + 1 + 2 * NB:2 * NB + 1 + 3 * NB]
        ssem = scratch[2 * NB + 1 + 3 * NB:]

        wid = lax.axis_index("s") * info.num_cores + lax.axis_index("c")
        row0 = wid * chunks

        @pl.when(lax.axis_index("s") == 0)
        def _():
            pltpu.sync_copy(pos_hbm, pos_sh)

        plsc.subcore_barrier()

        def x_slice(i):
            return x_hbm.at[pl.ds((row0 + i) * seqlen, seqlen)]

        def fetch(i, b):
            @pl.when(i < chunks)
            def _():
                @pl.when(i >= NB)
                def _():
                    pltpu.make_async_copy(
                        rows[b], out_hbm.at[row0 + i - NB], ssem[b]).wait()

                pltpu.async_copy(x_slice(i), idxs[b], isem[b])
                pltpu.async_copy(pos_sh, rows[b], nsem[b])

        def gather(i, b):
            @pl.when(jnp.logical_and(i >= 0, i < chunks))
            def _():
                pltpu.make_async_copy(x_slice(i), idxs[b], isem[b]).wait()
                pltpu.make_async_copy(pos_sh, rows[b], nsem[b]).wait()
                pltpu.async_copy(tok_hbm.at[idxs[b]], rows[b], gsem[b],
                                 add=True)

        def store(i, b):
            @pl.when(jnp.logical_and(i >= 0, i < chunks))
            def _():
                pltpu.make_async_copy(tok_hbm.at[idxs[b]], rows[b],
                                      gsem[b]).wait()
                pltpu.async_copy(rows[b], out_hbm.at[row0 + i], ssem[b])

        def visit_group(kk, _):
            for j in range(NB):
                v = NB * kk + j - 2
                fetch(v + 2, j)
                gather(v + 1, (j + 3) % NB)
                store(v, (j + 2) % NB)
            return 0

        lax.fori_loop(0, chunks // NB + 1, visit_group, 0)

        for b in range(NB):
            pltpu.make_async_copy(rows[b], out_hbm.at[row0], ssem[b]).wait()

    return k


def kernel(x, token_table, pos_table):
    batch, seqlen = x.shape
    vocab, embed = token_table.shape
    k = _make_sc_lookup(batch, seqlen, embed, vocab)
    return k(x.reshape(-1).astype(jnp.int32), token_table, pos_table)
